# baseline (speedup 1.0000x reference)
import functools

import jax
import jax.numpy as jnp
from jax import lax
from jax.experimental import pallas as pl
from jax.experimental.pallas import tpu as pltpu
from jax.experimental.pallas import tpu_sc as plsc
import numpy as np

N = 10000
E = 320000
D = 128
NW = 32
NPT = 320
NSUB = 8
NSR = NPT // NSUB
NB = 4
NPAD = NW * NPT
CAP = 1664
SCAP = 12800
K = 128
CH = 8000
GH = 25
HR = GH * 16 + 16
SENT = 3.0e38
DUMP = NPT
DELTA_LOG2 = float(np.log(2.0))


def _wid():
    return lax.axis_index("s") * 2 + lax.axis_index("c")


def _bucket_body(src_hbm, dst_hbm,
                 ind_hbm, counts_hbm, cnt_hbm, deg_hbm,
                 src_ch, dst_ch, *rest):
    srcs = rest[0:NSUB]
    dlocs = rest[NSUB:2 * NSUB]
    sloc_l, cnt_v, deg_v, crow = rest[2 * NSUB:]
    wid = _wid()
    lo = wid * NPT
    lanes = lax.broadcasted_iota(jnp.int32, (16,), 0)

    def zinit(r, _):
        z = jnp.zeros((16,), jnp.int32)
        cnt_v[pl.ds(r * 16, 16)] = z
        deg_v[pl.ds(r * 16, 16)] = z
        return 0
    lax.fori_loop(0, NPT + 1, zinit, 0)

    def chunk_body(ci, carry):
        pltpu.sync_copy(src_hbm.at[pl.ds(ci * CH, CH)], src_ch)
        pltpu.sync_copy(dst_hbm.at[pl.ds(ci * CH, CH)], dst_ch)

        def grp_body(g, carry2):
            ps = [jnp.minimum(p, CAP - 16) for p in carry2[:NSUB]]
            pS0 = jnp.minimum(carry2[NSUB], SCAP - 16)

            def vec_body(v, c2):
                sv = src_ch[pl.ds(g * (GH * 16) + v * 16, 16)]
                dv = dst_ch[pl.ds(g * (GH * 16) + v * 16, 16)]
                dl = dv - lo
                out = []
                for i in range(NSUB):
                    mi = (dl >= i * NSR) & (dl < (i + 1) * NSR)
                    o0 = c2[i][0]
                    plsc.store_compressed(srcs[i].at[pl.ds(o0, 16)],
                                          sv, mask=mi)
                    plsc.store_compressed(dlocs[i].at[pl.ds(o0, 16)],
                                          dl, mask=mi)
                    out.append(c2[i]
                               + plsc.all_reduce_population_count(mi))
                mS = (sv >= lo) & (sv < lo + NPT)
                s0 = c2[NSUB][0]
                plsc.store_compressed(sloc_l.at[pl.ds(s0, 16)],
                                      sv - lo, mask=mS)
                out.append(c2[NSUB]
                           + plsc.all_reduce_population_count(mS))
                return tuple(out)

            return lax.fori_loop(0, GH, vec_body, (*ps, pS0))

        return lax.fori_loop(0, CH // (16 * GH), grp_body, carry)

    zero = jnp.zeros((16,), jnp.int32)
    pend = lax.fori_loop(0, E // CH, chunk_body, (zero,) * (NSUB + 1))
    poss = [jnp.minimum(pend[i][0], CAP - 16) for i in range(NSUB)]
    posS = jnp.minimum(pend[NSUB][0], SCAP - 16)

    def pad_list(src_l, dloc_l, pos, quantum):
        padded = jnp.maximum(((pos + quantum - 1) // quantum) * quantum,
                             quantum)
        wfirst = lax.shift_left(lax.shift_right_logical(pos, 4), 4)

        def pad_body(i, _):
            off = wfirst + i * 16
            sl = pl.ds(off, 16)
            keep = lanes < (pos - off)
            src_l[sl] = jnp.where(keep, src_l[sl], 0)
            dloc_l[sl] = jnp.where(keep, dloc_l[sl], DUMP)
            return 0
        lax.fori_loop(0, lax.shift_right_logical(padded - wfirst, 4),
                      pad_body, 0)
        return padded

    paddeds = [pad_list(srcs[i], dlocs[i], poss[i], K) for i in range(NSUB)]

    swfirst = lax.shift_left(lax.shift_right_logical(posS, 4), 4)
    skeep = lanes < (posS - swfirst)
    ssl = pl.ds(swfirst, 16)
    sloc_l[ssl] = jnp.where(skeep, sloc_l[ssl], DUMP)
    spadded = lax.shift_left(lax.shift_right_logical(posS + 15, 4), 4)

    one = jnp.ones((16,), jnp.int32)

    def hist(list_ref, hist_ref, padded16):
        def hbody(w, _):
            vec = list_ref[pl.ds(w * 16, 16)]
            for lane in range(16):
                dd = vec[lane] * 16
                sl = pl.ds(dd, 16)
                hist_ref[sl] = hist_ref[sl] + one
            return 0
        lax.fori_loop(0, lax.shift_right_logical(padded16, 4), hbody, 0)

    for i in range(NSUB):
        hist(dlocs[i], cnt_v, paddeds[i])
    hist(sloc_l, deg_v, spadded)

    for i in range(NSUB):
        crow[i, :] = jnp.zeros((16,), jnp.int32) + paddeds[i]
        pltpu.sync_copy(srcs[i].at[pl.ds(0, CAP)], ind_hbm.at[wid, i, 0])
        pltpu.sync_copy(dlocs[i].at[pl.ds(0, CAP)], ind_hbm.at[wid, i, 1])
    pltpu.sync_copy(crow, counts_hbm.at[wid])
    pltpu.sync_copy(cnt_v.at[pl.ds(0, NPT * 16)],
                    cnt_hbm.at[pl.ds(lo * 16, NPT * 16)])
    pltpu.sync_copy(deg_v.at[pl.ds(0, NPT * 16)],
                    deg_hbm.at[pl.ds(lo * 16, NPT * 16)])


def _bucket(src, dst):
    mesh = plsc.VectorSubcoreMesh(core_axis_name="c", subcore_axis_name="s")
    f = functools.partial(
        pl.kernel,
        mesh=mesh,
        compiler_params=pltpu.CompilerParams(needs_layout_passes=False),
        out_type=(
            jax.ShapeDtypeStruct((NW, NSUB, 2, CAP), jnp.int32),
            jax.ShapeDtypeStruct((NW, NSUB, 16), jnp.int32),
            jax.ShapeDtypeStruct((NPAD * 16,), jnp.int32),
            jax.ShapeDtypeStruct((NPAD * 16,), jnp.int32),
        ),
        scratch_types=(
            [pltpu.VMEM((CH,), jnp.int32)] * 2
            + [pltpu.VMEM((CAP + HR,), jnp.int32)] * (2 * NSUB)
            + [
                pltpu.VMEM((SCAP + HR,), jnp.int32),
                pltpu.VMEM(((NPT + 1) * 16,), jnp.int32),
                pltpu.VMEM(((NPT + 1) * 16,), jnp.int32),
                pltpu.VMEM((NSUB, 16), jnp.int32),
            ]
        ),
    )
    return f(_bucket_body)(src, dst)


def _accum_body(h_hbm, ind_hbm, counts_hbm, accs_hbm, *rest):
    sums = rest[0:NB]
    ssqs = rest[NB:2 * NB]
    maxs = rest[2 * NB:3 * NB]
    mins = rest[3 * NB:4 * NB]
    msg_v, indb, crow, sem = rest[4 * NB:]
    wid = _wid()
    pltpu.sync_copy(counts_hbm.at[wid], crow)

    z = jnp.zeros((16,), jnp.float32)
    neg = jnp.full((16,), -SENT, jnp.float32)
    pos = jnp.full((16,), SENT, jnp.float32)

    def one_pass(p, _):
        padded = crow[p, :][0]
        nchunk = lax.shift_right_logical(padded, 7)
        sub_lo = p * NSR

        def ainit(r, _2):
            for c in range(D // 16):
                sl = pl.ds(c * 16, 16)
                for b in range(NB):
                    sums[b][r, sl] = z
                    ssqs[b][r, sl] = z
                    maxs[b][r, sl] = neg
                    mins[b][r, sl] = pos
            return 0
        lax.fori_loop(0, NSR + 1, ainit, 0)

        def chunk(j, _2):
            pltpu.sync_copy(ind_hbm.at[wid, p, :, pl.ds(j * K, K)], indb)
            pltpu.async_copy(h_hbm.at[indb.at[0]], msg_v, sem).wait()

            def edge(w, _3):
                dvec = indb[1, pl.ds(w * 16, 16)]
                dvec = jnp.minimum(dvec - sub_lo, NSR)
                for lane in range(16):
                    b = lane % NB
                    dl = dvec[lane]
                    e = w * 16 + lane
                    for c in range(D // 16):
                        sl = pl.ds(c * 16, 16)
                        m = msg_v[e, sl]
                        plsc.addupdate(sums[b].at[dl, sl], m)
                        plsc.addupdate(ssqs[b].at[dl, sl], m * m)
                        mx = maxs[b][dl, sl]
                        maxs[b][dl, sl] = jnp.maximum(mx, m)
                        mn = mins[b][dl, sl]
                        mins[b][dl, sl] = jnp.minimum(mn, m)
                return 0
            lax.fori_loop(0, K // 16, edge, 0)
            return 0
        lax.fori_loop(0, nchunk, chunk, 0)

        out_lo = wid * NPT + sub_lo
        for a, (banks, op) in enumerate((
                (sums, jnp.add), (ssqs, jnp.add),
                (maxs, jnp.maximum), (mins, jnp.minimum))):
            def merge(r, _2):
                for c in range(D // 16):
                    sl = pl.ds(c * 16, 16)
                    v = op(op(banks[0][r, sl], banks[1][r, sl]),
                           op(banks[2][r, sl], banks[3][r, sl]))
                    msg_v[r, sl] = v
                return 0
            lax.fori_loop(0, NSR, merge, 0)
            pltpu.sync_copy(msg_v.at[pl.ds(0, NSR)],
                            accs_hbm.at[a, pl.ds(out_lo, NSR)])
        return 0
    lax.fori_loop(0, NSUB, one_pass, 0)


def _accum(h, ind, counts):
    mesh = plsc.VectorSubcoreMesh(core_axis_name="c", subcore_axis_name="s")
    f = functools.partial(
        pl.kernel,
        mesh=mesh,
        compiler_params=pltpu.CompilerParams(needs_layout_passes=False),
        out_type=jax.ShapeDtypeStruct((4, NPAD, D), jnp.float32),
        scratch_types=(
            [pltpu.VMEM((NSR + 1, D), jnp.float32)] * (4 * NB)
            + [
                pltpu.VMEM((K, D), jnp.float32),
                pltpu.VMEM((2, K), jnp.int32),
                pltpu.VMEM((NSUB, 16), jnp.int32),
                pltpu.SemaphoreType.DMA,
            ]
        ),
    )
    return f(_accum_body)(h, ind, counts)


def _tc_body(acc_ref, cnt_ref, deg_ref, h_ref, w_ref, b_ref, o_ref):
    cnt = cnt_ref[...].astype(jnp.float32)
    ne = cnt > 0.0
    inv = 1.0 / jnp.maximum(cnt, 1.0)
    deg = jnp.maximum(deg_ref[...].astype(jnp.float32), 1.0)
    la = jnp.log(deg + 1.0)
    amp = la * (1.0 / DELTA_LOG2)
    att = DELTA_LOG2 / la
    s = acc_ref[0]
    s2 = acc_ref[1]
    mx = jnp.where(ne, acc_ref[2], 0.0)
    mn = jnp.where(ne, acc_ref[3], 0.0)
    mean = s * inv
    var = jnp.maximum(s2 * inv - mean * mean, 0.0)
    std = jnp.sqrt(var + 1e-5)
    agg = jnp.concatenate([mean, mx, mn, std], axis=1)
    w = w_ref[...]
    o = jnp.dot(agg, w[0:4 * D], preferred_element_type=jnp.float32)
    o += amp * jnp.dot(agg, w[4 * D:8 * D], preferred_element_type=jnp.float32)
    o += att * jnp.dot(agg, w[8 * D:12 * D], preferred_element_type=jnp.float32)
    o_ref[...] = o + b_ref[...] + h_ref[...]


def _tc_layer(accs, cnt2, deg2, h, W, b2):
    R = 512
    grid = (NPAD // R,)
    return pl.pallas_call(
        _tc_body,
        grid=grid,
        in_specs=[
            pl.BlockSpec((4, R, D), lambda i: (0, i, 0)),
            pl.BlockSpec((R, 1), lambda i: (i, 0)),
            pl.BlockSpec((R, 1), lambda i: (i, 0)),
            pl.BlockSpec((R, D), lambda i: (i, 0)),
            pl.BlockSpec((12 * D, D), lambda i: (0, 0)),
            pl.BlockSpec((1, D), lambda i: (0, 0)),
        ],
        out_specs=pl.BlockSpec((R, D), lambda i: (i, 0)),
        out_shape=jax.ShapeDtypeStruct((NPAD, D), jnp.float32),
    )(accs, cnt2, deg2, h, W, b2)


def kernel(x, edge_index, W0, b0, W1, b1, W2, b2):
    src = edge_index[0]
    dst = edge_index[1]
    ind, counts, cnt_i, deg_i = _bucket(src, dst)
    cnt2 = cnt_i.reshape(NPAD, 16)[:, :1]
    deg2 = deg_i.reshape(NPAD, 16)[:, :1]
    h = jnp.pad(x, ((0, NPAD - N), (0, 0)))
    for W, b in ((W0, b0), (W1, b1), (W2, b2)):
        accs = _accum(h, ind, counts)
        h = _tc_layer(accs, cnt2, deg2, h, W, b.reshape(1, D))
    return h[:N]


# speedup vs baseline: 1.1929x; 1.1929x over previous
import functools

import jax
import jax.numpy as jnp
from jax import lax
from jax.experimental import pallas as pl
from jax.experimental.pallas import tpu as pltpu
from jax.experimental.pallas import tpu_sc as plsc
import numpy as np

N = 10000
E = 320000
D = 128
NW = 32
NPT = 320
NSUB = 4
NSR = NPT // NSUB
NB = 2
NPAD = NW * NPT
CAP = 2944
SCAP = 12800
K = 128
CH = 8000
GH = 25
HR = GH * 16 + 16
SENT = 3.0e38
DUMP = NPT
DELTA_LOG2 = float(np.log(2.0))


def _wid():
    return lax.axis_index("s") * 2 + lax.axis_index("c")


def _bucket_body(src_hbm, dst_hbm,
                 ind_hbm, counts_hbm, cnt_hbm, deg_hbm,
                 src_ch, dst_ch, *rest):
    srcs = rest[0:NSUB]
    dlocs = rest[NSUB:2 * NSUB]
    sloc_l, cnt_v, deg_v, crow = rest[2 * NSUB:]
    wid = _wid()
    lo = wid * NPT
    lanes = lax.broadcasted_iota(jnp.int32, (16,), 0)

    def zinit(r, _):
        z = jnp.zeros((16,), jnp.int32)
        cnt_v[pl.ds(r * 16, 16)] = z
        deg_v[pl.ds(r * 16, 16)] = z
        return 0
    lax.fori_loop(0, NPT + 1, zinit, 0)

    def chunk_body(ci, carry):
        pltpu.sync_copy(src_hbm.at[pl.ds(ci * CH, CH)], src_ch)
        pltpu.sync_copy(dst_hbm.at[pl.ds(ci * CH, CH)], dst_ch)

        def grp_body(g, carry2):
            ps = [jnp.minimum(p, CAP - 16) for p in carry2[:NSUB]]
            pS0 = jnp.minimum(carry2[NSUB], SCAP - 16)

            def vec_body(v, c2):
                sv = src_ch[pl.ds(g * (GH * 16) + v * 16, 16)]
                dv = dst_ch[pl.ds(g * (GH * 16) + v * 16, 16)]
                dl = dv - lo
                out = []
                for i in range(NSUB):
                    mi = (dl >= i * NSR) & (dl < (i + 1) * NSR)
                    o0 = c2[i][0]
                    plsc.store_compressed(srcs[i].at[pl.ds(o0, 16)],
                                          sv, mask=mi)
                    plsc.store_compressed(dlocs[i].at[pl.ds(o0, 16)],
                                          dl, mask=mi)
                    out.append(c2[i]
                               + plsc.all_reduce_population_count(mi))
                mS = (sv >= lo) & (sv < lo + NPT)
                s0 = c2[NSUB][0]
                plsc.store_compressed(sloc_l.at[pl.ds(s0, 16)],
                                      sv - lo, mask=mS)
                out.append(c2[NSUB]
                           + plsc.all_reduce_population_count(mS))
                return tuple(out)

            return lax.fori_loop(0, GH, vec_body, (*ps, pS0))

        return lax.fori_loop(0, CH // (16 * GH), grp_body, carry)

    zero = jnp.zeros((16,), jnp.int32)
    pend = lax.fori_loop(0, E // CH, chunk_body, (zero,) * (NSUB + 1))
    poss = [jnp.minimum(pend[i][0], CAP - 16) for i in range(NSUB)]
    posS = jnp.minimum(pend[NSUB][0], SCAP - 16)

    def pad_list(src_l, dloc_l, pos, quantum):
        padded = jnp.maximum(((pos + quantum - 1) // quantum) * quantum,
                             quantum)
        wfirst = lax.shift_left(lax.shift_right_logical(pos, 4), 4)

        def pad_body(i, _):
            off = wfirst + i * 16
            sl = pl.ds(off, 16)
            keep = lanes < (pos - off)
            src_l[sl] = jnp.where(keep, src_l[sl], 0)
            dloc_l[sl] = jnp.where(keep, dloc_l[sl], DUMP)
            return 0
        lax.fori_loop(0, lax.shift_right_logical(padded - wfirst, 4),
                      pad_body, 0)
        return padded

    paddeds = [pad_list(srcs[i], dlocs[i], poss[i], K) for i in range(NSUB)]

    swfirst = lax.shift_left(lax.shift_right_logical(posS, 4), 4)
    skeep = lanes < (posS - swfirst)
    ssl = pl.ds(swfirst, 16)
    sloc_l[ssl] = jnp.where(skeep, sloc_l[ssl], DUMP)
    spadded = lax.shift_left(lax.shift_right_logical(posS + 15, 4), 4)

    one = jnp.ones((16,), jnp.int32)

    def hist(list_ref, hist_ref, padded16):
        def hbody(w, _):
            vec = list_ref[pl.ds(w * 16, 16)]
            for lane in range(16):
                dd = vec[lane] * 16
                sl = pl.ds(dd, 16)
                hist_ref[sl] = hist_ref[sl] + one
            return 0
        lax.fori_loop(0, lax.shift_right_logical(padded16, 4), hbody, 0)

    for i in range(NSUB):
        hist(dlocs[i], cnt_v, paddeds[i])
    hist(sloc_l, deg_v, spadded)

    for i in range(NSUB):
        crow[i, :] = jnp.zeros((16,), jnp.int32) + paddeds[i]
        pltpu.sync_copy(srcs[i].at[pl.ds(0, CAP)], ind_hbm.at[wid, i, 0])
        pltpu.sync_copy(dlocs[i].at[pl.ds(0, CAP)], ind_hbm.at[wid, i, 1])
    pltpu.sync_copy(crow, counts_hbm.at[wid])
    pltpu.sync_copy(cnt_v.at[pl.ds(0, NPT * 16)],
                    cnt_hbm.at[pl.ds(lo * 16, NPT * 16)])
    pltpu.sync_copy(deg_v.at[pl.ds(0, NPT * 16)],
                    deg_hbm.at[pl.ds(lo * 16, NPT * 16)])


def _bucket(src, dst):
    mesh = plsc.VectorSubcoreMesh(core_axis_name="c", subcore_axis_name="s")
    f = functools.partial(
        pl.kernel,
        mesh=mesh,
        compiler_params=pltpu.CompilerParams(needs_layout_passes=False),
        out_type=(
            jax.ShapeDtypeStruct((NW, NSUB, 2, CAP), jnp.int32),
            jax.ShapeDtypeStruct((NW, NSUB, 16), jnp.int32),
            jax.ShapeDtypeStruct((NPAD * 16,), jnp.int32),
            jax.ShapeDtypeStruct((NPAD * 16,), jnp.int32),
        ),
        scratch_types=(
            [pltpu.VMEM((CH,), jnp.int32)] * 2
            + [pltpu.VMEM((CAP + HR,), jnp.int32)] * (2 * NSUB)
            + [
                pltpu.VMEM((SCAP + HR,), jnp.int32),
                pltpu.VMEM(((NPT + 1) * 16,), jnp.int32),
                pltpu.VMEM(((NPT + 1) * 16,), jnp.int32),
                pltpu.VMEM((NSUB, 16), jnp.int32),
            ]
        ),
    )
    return f(_bucket_body)(src, dst)


def _accum_body(h_hbm, ind_hbm, counts_hbm, accs_hbm, *rest):
    sums = rest[0:NB]
    ssqs = rest[NB:2 * NB]
    maxs = rest[2 * NB:3 * NB]
    mins = rest[3 * NB:4 * NB]
    msg_v, indb, crow, sem = rest[4 * NB:]
    wid = _wid()
    pltpu.sync_copy(counts_hbm.at[wid], crow)

    z = jnp.zeros((16,), jnp.float32)
    neg = jnp.full((16,), -SENT, jnp.float32)
    pos = jnp.full((16,), SENT, jnp.float32)

    def one_pass(p, _):
        padded = crow[p, :][0]
        nchunk = lax.shift_right_logical(padded, 7)
        sub_lo = p * NSR

        def ainit(r, _2):
            for c in range(D // 16):
                sl = pl.ds(c * 16, 16)
                for b in range(NB):
                    sums[b][r, sl] = z
                    ssqs[b][r, sl] = z
                    maxs[b][r, sl] = neg
                    mins[b][r, sl] = pos
            return 0
        lax.fori_loop(0, NSR + 1, ainit, 0)

        def chunk(j, _2):
            pltpu.sync_copy(ind_hbm.at[wid, p, :, pl.ds(j * K, K)], indb)
            pltpu.async_copy(h_hbm.at[indb.at[0]], msg_v, sem).wait()

            def edge(w, _3):
                dvec = indb[1, pl.ds(w * 16, 16)]
                dvec = jnp.minimum(dvec - sub_lo, NSR)
                for lane in range(16):
                    b = lane % NB
                    dl = dvec[lane]
                    e = w * 16 + lane
                    for c in range(D // 16):
                        sl = pl.ds(c * 16, 16)
                        m = msg_v[e, sl]
                        plsc.addupdate(sums[b].at[dl, sl], m)
                        plsc.addupdate(ssqs[b].at[dl, sl], m * m)
                        mx = maxs[b][dl, sl]
                        maxs[b][dl, sl] = jnp.maximum(mx, m)
                        mn = mins[b][dl, sl]
                        mins[b][dl, sl] = jnp.minimum(mn, m)
                return 0
            lax.fori_loop(0, K // 16, edge, 0)
            return 0
        lax.fori_loop(0, nchunk, chunk, 0)

        out_lo = wid * NPT + sub_lo
        for a, (banks, op) in enumerate((
                (sums, jnp.add), (ssqs, jnp.add),
                (maxs, jnp.maximum), (mins, jnp.minimum))):
            def merge(r, _2):
                for c in range(D // 16):
                    sl = pl.ds(c * 16, 16)
                    v = banks[0][r, sl]
                    for b in range(1, NB):
                        v = op(v, banks[b][r, sl])
                    msg_v[r, sl] = v
                return 0
            lax.fori_loop(0, NSR, merge, 0)
            pltpu.sync_copy(msg_v.at[pl.ds(0, NSR)],
                            accs_hbm.at[a, pl.ds(out_lo, NSR)])
        return 0
    lax.fori_loop(0, NSUB, one_pass, 0)


def _accum(h, ind, counts):
    mesh = plsc.VectorSubcoreMesh(core_axis_name="c", subcore_axis_name="s")
    f = functools.partial(
        pl.kernel,
        mesh=mesh,
        compiler_params=pltpu.CompilerParams(needs_layout_passes=False),
        out_type=jax.ShapeDtypeStruct((4, NPAD, D), jnp.float32),
        scratch_types=(
            [pltpu.VMEM((NSR + 1, D), jnp.float32)] * (4 * NB)
            + [
                pltpu.VMEM((K, D), jnp.float32),
                pltpu.VMEM((2, K), jnp.int32),
                pltpu.VMEM((NSUB, 16), jnp.int32),
                pltpu.SemaphoreType.DMA,
            ]
        ),
    )
    return f(_accum_body)(h, ind, counts)


def _tc_body(acc_ref, cnt_ref, deg_ref, h_ref, w_ref, b_ref, o_ref):
    cnt = cnt_ref[...].astype(jnp.float32)
    ne = cnt > 0.0
    inv = 1.0 / jnp.maximum(cnt, 1.0)
    deg = jnp.maximum(deg_ref[...].astype(jnp.float32), 1.0)
    la = jnp.log(deg + 1.0)
    amp = la * (1.0 / DELTA_LOG2)
    att = DELTA_LOG2 / la
    s = acc_ref[0]
    s2 = acc_ref[1]
    mx = jnp.where(ne, acc_ref[2], 0.0)
    mn = jnp.where(ne, acc_ref[3], 0.0)
    mean = s * inv
    var = jnp.maximum(s2 * inv - mean * mean, 0.0)
    std = jnp.sqrt(var + 1e-5)
    agg = jnp.concatenate([mean, mx, mn, std], axis=1)
    w = w_ref[...]
    o = jnp.dot(agg, w[0:4 * D], preferred_element_type=jnp.float32)
    o += amp * jnp.dot(agg, w[4 * D:8 * D], preferred_element_type=jnp.float32)
    o += att * jnp.dot(agg, w[8 * D:12 * D], preferred_element_type=jnp.float32)
    o_ref[...] = o + b_ref[...] + h_ref[...]


def _tc_layer(accs, cnt2, deg2, h, W, b2):
    R = 512
    grid = (NPAD // R,)
    return pl.pallas_call(
        _tc_body,
        grid=grid,
        in_specs=[
            pl.BlockSpec((4, R, D), lambda i: (0, i, 0)),
            pl.BlockSpec((R, 1), lambda i: (i, 0)),
            pl.BlockSpec((R, 1), lambda i: (i, 0)),
            pl.BlockSpec((R, D), lambda i: (i, 0)),
            pl.BlockSpec((12 * D, D), lambda i: (0, 0)),
            pl.BlockSpec((1, D), lambda i: (0, 0)),
        ],
        out_specs=pl.BlockSpec((R, D), lambda i: (i, 0)),
        out_shape=jax.ShapeDtypeStruct((NPAD, D), jnp.float32),
    )(accs, cnt2, deg2, h, W, b2)


def kernel(x, edge_index, W0, b0, W1, b1, W2, b2):
    src = edge_index[0]
    dst = edge_index[1]
    ind, counts, cnt_i, deg_i = _bucket(src, dst)
    cnt2 = cnt_i.reshape(NPAD, 16)[:, :1]
    deg2 = deg_i.reshape(NPAD, 16)[:, :1]
    h = jnp.pad(x, ((0, NPAD - N), (0, 0)))
    for W, b in ((W0, b0), (W1, b1), (W2, b2)):
        accs = _accum(h, ind, counts)
        h = _tc_layer(accs, cnt2, deg2, h, W, b.reshape(1, D))
    return h[:N]


# speedup vs baseline: 1.2207x; 1.0233x over previous
import functools

import jax
import jax.numpy as jnp
from jax import lax
from jax.experimental import pallas as pl
from jax.experimental.pallas import tpu as pltpu
from jax.experimental.pallas import tpu_sc as plsc
import numpy as np

N = 10000
E = 320000
D = 128
NW = 32
NPT = 320
NSR = 160
NPAD = NW * NPT
CAP = 6400
SCAP = 12800
K = 128
CH = 8000
SENT = 3.0e38
DUMP = 2 * NSR
DELTA_LOG2 = float(np.log(2.0))


def _wid():
    return lax.axis_index("s") * 2 + lax.axis_index("c")


def _bucket_body(src_hbm, dst_hbm,
                 ind_hbm, counts_hbm, cnt_hbm, deg_hbm,
                 src_ch, dst_ch, srcA, dlocA, srcB, dlocB, sloc_l,
                 cnt_v, deg_v, crow):
    wid = _wid()
    lo = wid * NPT
    mid = lo + NSR
    lanes = lax.broadcasted_iota(jnp.int32, (16,), 0)

    def zinit(r, _):
        z = jnp.zeros((16,), jnp.int32)
        cnt_v[pl.ds(r * 16, 16)] = z
        deg_v[pl.ds(r * 16, 16)] = z
        return 0
    lax.fori_loop(0, NPT + 1, zinit, 0)

    def chunk_body(ci, carry):
        pltpu.sync_copy(src_hbm.at[pl.ds(ci * CH, CH)], src_ch)
        pltpu.sync_copy(dst_hbm.at[pl.ds(ci * CH, CH)], dst_ch)
        pA0, pB0, pS0 = carry
        pA0 = jnp.minimum(pA0, CAP - 16)
        pB0 = jnp.minimum(pB0, CAP - 16)
        pS0 = jnp.minimum(pS0, SCAP - 16)

        def vec_body(v, c2):
            pA, pB, pS = c2
            sv = src_ch[pl.ds(v * 16, 16)]
            dv = dst_ch[pl.ds(v * 16, 16)]
            dl = dv - lo
            mA = (dv >= lo) & (dv < mid)
            mB = (dv >= mid) & (dv < lo + NPT)
            mS = (sv >= lo) & (sv < lo + NPT)
            a0 = pA[0]
            b0 = pB[0]
            s0 = pS[0]
            plsc.store_compressed(srcA.at[pl.ds(a0, 16)], sv, mask=mA)
            plsc.store_compressed(dlocA.at[pl.ds(a0, 16)], dl, mask=mA)
            plsc.store_compressed(srcB.at[pl.ds(b0, 16)], sv, mask=mB)
            plsc.store_compressed(dlocB.at[pl.ds(b0, 16)], dl, mask=mB)
            plsc.store_compressed(sloc_l.at[pl.ds(s0, 16)], sv - lo, mask=mS)
            pA = pA + plsc.all_reduce_population_count(mA)
            pB = pB + plsc.all_reduce_population_count(mB)
            pS = pS + plsc.all_reduce_population_count(mS)
            return pA, pB, pS

        return lax.fori_loop(0, CH // 16, vec_body, (pA0, pB0, pS0))

    zero = jnp.zeros((16,), jnp.int32)
    pAv, pBv, pSv = lax.fori_loop(0, E // CH, chunk_body, (zero, zero, zero))
    posA = jnp.minimum(pAv[0], CAP - 16)
    posB = jnp.minimum(pBv[0], CAP - 16)
    posS = jnp.minimum(pSv[0], SCAP - 16)

    def pad_list(src_l, dloc_l, pos, quantum):
        padded = jnp.maximum(((pos + quantum - 1) // quantum) * quantum,
                             quantum)
        wfirst = lax.shift_left(lax.shift_right_logical(pos, 4), 4)

        def pad_body(i, _):
            off = wfirst + i * 16
            sl = pl.ds(off, 16)
            keep = lanes < (pos - off)
            src_l[sl] = jnp.where(keep, src_l[sl], 0)
            dloc_l[sl] = jnp.where(keep, dloc_l[sl], DUMP)
            return 0
        lax.fori_loop(0, lax.shift_right_logical(padded - wfirst, 4),
                      pad_body, 0)
        return padded

    paddedA = pad_list(srcA, dlocA, posA, K)
    paddedB = pad_list(srcB, dlocB, posB, K)

    swfirst = lax.shift_left(lax.shift_right_logical(posS, 4), 4)
    skeep = lanes < (posS - swfirst)
    ssl = pl.ds(swfirst, 16)
    sloc_l[ssl] = jnp.where(skeep, sloc_l[ssl], DUMP)
    spadded = lax.shift_left(lax.shift_right_logical(posS + 15, 4), 4)

    one = jnp.ones((16,), jnp.int32)

    def hist(list_ref, hist_ref, padded16):
        def hbody(w, _):
            vec = list_ref[pl.ds(w * 16, 16)]
            for lane in range(16):
                dd = vec[lane] * 16
                sl = pl.ds(dd, 16)
                hist_ref[sl] = hist_ref[sl] + one
            return 0
        lax.fori_loop(0, lax.shift_right_logical(padded16, 4), hbody, 0)

    hist(dlocA, cnt_v, paddedA)
    hist(dlocB, cnt_v, paddedB)
    hist(sloc_l, deg_v, spadded)

    crow[0, :] = jnp.zeros((16,), jnp.int32) + paddedA
    crow[1, :] = jnp.zeros((16,), jnp.int32) + paddedB
    pltpu.sync_copy(srcA.at[pl.ds(0, CAP)], ind_hbm.at[wid, 0, 0])
    pltpu.sync_copy(srcB.at[pl.ds(0, CAP)], ind_hbm.at[wid, 1, 0])
    pltpu.sync_copy(dlocA.at[pl.ds(0, CAP)], ind_hbm.at[wid, 0, 1])
    pltpu.sync_copy(dlocB.at[pl.ds(0, CAP)], ind_hbm.at[wid, 1, 1])
    pltpu.sync_copy(crow, counts_hbm.at[wid])
    pltpu.sync_copy(cnt_v.at[pl.ds(0, NPT * 16)],
                    cnt_hbm.at[pl.ds(lo * 16, NPT * 16)])
    pltpu.sync_copy(deg_v.at[pl.ds(0, NPT * 16)],
                    deg_hbm.at[pl.ds(lo * 16, NPT * 16)])


def _bucket(src, dst):
    mesh = plsc.VectorSubcoreMesh(core_axis_name="c", subcore_axis_name="s")
    f = functools.partial(
        pl.kernel,
        mesh=mesh,
        compiler_params=pltpu.CompilerParams(needs_layout_passes=False),
        out_type=(
            jax.ShapeDtypeStruct((NW, 2, 2, CAP), jnp.int32),
            jax.ShapeDtypeStruct((NW, 2, 16), jnp.int32),
            jax.ShapeDtypeStruct((NPAD * 16,), jnp.int32),
            jax.ShapeDtypeStruct((NPAD * 16,), jnp.int32),
        ),
        scratch_types=[
            pltpu.VMEM((CH,), jnp.int32),
            pltpu.VMEM((CH,), jnp.int32),
            pltpu.VMEM((CAP + CH,), jnp.int32),
            pltpu.VMEM((CAP + CH,), jnp.int32),
            pltpu.VMEM((CAP + CH,), jnp.int32),
            pltpu.VMEM((CAP + CH,), jnp.int32),
            pltpu.VMEM((SCAP + CH,), jnp.int32),
            pltpu.VMEM(((NPT + 1) * 16,), jnp.int32),
            pltpu.VMEM(((NPT + 1) * 16,), jnp.int32),
            pltpu.VMEM((2, 16), jnp.int32),
        ],
    )
    return f(_bucket_body)(src, dst)


def _accum_body(h_hbm, ind_hbm, counts_hbm, accs_hbm,
                sum_v, ssq_v, max_v, min_v, msg_v, indb, crow, sem):
    wid = _wid()
    pltpu.sync_copy(counts_hbm.at[wid], crow)

    z = jnp.zeros((16,), jnp.float32)
    neg = jnp.full((16,), -SENT, jnp.float32)
    pos = jnp.full((16,), SENT, jnp.float32)

    for p in range(2):
        padded = crow[p, :][0]
        nchunk = lax.shift_right_logical(padded, 7)
        sub_lo = p * NSR

        def ainit(r, _):
            for c in range(D // 16):
                sl = pl.ds(c * 16, 16)
                sum_v[r, sl] = z
                ssq_v[r, sl] = z
                max_v[r, sl] = neg
                min_v[r, sl] = pos
            return 0
        lax.fori_loop(0, NSR + 1, ainit, 0)

        def chunk(j, _):
            pltpu.sync_copy(ind_hbm.at[wid, p, :, pl.ds(j * K, K)], indb)
            pltpu.async_copy(h_hbm.at[indb.at[0]], msg_v, sem).wait()

            def edge(w, _):
                dvec = indb[1, pl.ds(w * 16, 16)]
                dvec = jnp.minimum(dvec - sub_lo, NSR)
                for lane in range(16):
                    dl = dvec[lane]
                    e = w * 16 + lane
                    for c in range(D // 16):
                        sl = pl.ds(c * 16, 16)
                        m = msg_v[e, sl]
                        plsc.addupdate(sum_v.at[dl, sl], m)
                        plsc.addupdate(ssq_v.at[dl, sl], m * m)
                        mx = max_v[dl, sl]
                        max_v[dl, sl] = jnp.maximum(mx, m)
                        mn = min_v[dl, sl]
                        min_v[dl, sl] = jnp.minimum(mn, m)
                return 0
            lax.fori_loop(0, K // 16, edge, 0)
            return 0
        lax.fori_loop(0, nchunk, chunk, 0)

        out_lo = wid * NPT + sub_lo
        for a, ref in enumerate((sum_v, ssq_v, max_v, min_v)):
            pltpu.sync_copy(ref.at[pl.ds(0, NSR)],
                            accs_hbm.at[a, pl.ds(out_lo, NSR)])


def _accum(h, ind, counts):
    mesh = plsc.VectorSubcoreMesh(core_axis_name="c", subcore_axis_name="s")
    f = functools.partial(
        pl.kernel,
        mesh=mesh,
        compiler_params=pltpu.CompilerParams(needs_layout_passes=False),
        out_type=jax.ShapeDtypeStruct((4, NPAD, D), jnp.float32),
        scratch_types=[
            pltpu.VMEM((NSR + 1, D), jnp.float32),
            pltpu.VMEM((NSR + 1, D), jnp.float32),
            pltpu.VMEM((NSR + 1, D), jnp.float32),
            pltpu.VMEM((NSR + 1, D), jnp.float32),
            pltpu.VMEM((K, D), jnp.float32),
            pltpu.VMEM((2, K), jnp.int32),
            pltpu.VMEM((2, 16), jnp.int32),
            pltpu.SemaphoreType.DMA,
        ],
    )
    return f(_accum_body)(h, ind, counts)


def _tc_body(acc_ref, cnt_ref, deg_ref, h_ref, w_ref, b_ref, o_ref):
    cnt = cnt_ref[...].astype(jnp.float32)
    ne = cnt > 0.0
    inv = 1.0 / jnp.maximum(cnt, 1.0)
    deg = jnp.maximum(deg_ref[...].astype(jnp.float32), 1.0)
    la = jnp.log(deg + 1.0)
    amp = la * (1.0 / DELTA_LOG2)
    att = DELTA_LOG2 / la
    s = acc_ref[0]
    s2 = acc_ref[1]
    mx = jnp.where(ne, acc_ref[2], 0.0)
    mn = jnp.where(ne, acc_ref[3], 0.0)
    mean = s * inv
    var = jnp.maximum(s2 * inv - mean * mean, 0.0)
    std = jnp.sqrt(var + 1e-5)
    agg = jnp.concatenate([mean, mx, mn, std], axis=1)
    w = w_ref[...]
    o = jnp.dot(agg, w[0:4 * D], preferred_element_type=jnp.float32)
    o += amp * jnp.dot(agg, w[4 * D:8 * D], preferred_element_type=jnp.float32)
    o += att * jnp.dot(agg, w[8 * D:12 * D], preferred_element_type=jnp.float32)
    o_ref[...] = o + b_ref[...] + h_ref[...]


def _tc_layer(accs, cnt2, deg2, h, W, b2):
    R = 512
    grid = (NPAD // R,)
    return pl.pallas_call(
        _tc_body,
        grid=grid,
        in_specs=[
            pl.BlockSpec((4, R, D), lambda i: (0, i, 0)),
            pl.BlockSpec((R, 1), lambda i: (i, 0)),
            pl.BlockSpec((R, 1), lambda i: (i, 0)),
            pl.BlockSpec((R, D), lambda i: (i, 0)),
            pl.BlockSpec((12 * D, D), lambda i: (0, 0)),
            pl.BlockSpec((1, D), lambda i: (0, 0)),
        ],
        out_specs=pl.BlockSpec((R, D), lambda i: (i, 0)),
        out_shape=jax.ShapeDtypeStruct((NPAD, D), jnp.float32),
    )(accs, cnt2, deg2, h, W, b2)


def kernel(x, edge_index, W0, b0, W1, b1, W2, b2):
    src = edge_index[0]
    dst = edge_index[1]
    ind, counts, cnt_i, deg_i = _bucket(src, dst)
    cnt2 = cnt_i.reshape(NPAD, 16)[:, :1]
    deg2 = deg_i.reshape(NPAD, 16)[:, :1]
    h = jnp.pad(x, ((0, NPAD - N), (0, 0)))
    for W, b in ((W0, b0), (W1, b1), (W2, b2)):
        accs = _accum(h, ind, counts)
        h = _tc_layer(accs, cnt2, deg2, h, W, b.reshape(1, D))
    return h[:N]
